# Initial kernel scaffold; baseline (speedup 1.0000x reference)
import jax
import jax.numpy as jnp
from jax.experimental import pallas as pl


def kernel(input, edge_index, W_conv, b_conv, W1, b1, W2, b2, W3, b3):
    raise NotImplementedError("write your pallas kernel here")


# speedup vs baseline: 99.1901x; 99.1901x over previous
import functools

import jax
import jax.numpy as jnp
from jax.experimental import pallas as pl
from jax.experimental.pallas import tpu as pltpu

_THRESH = 0.5
_DECAY = 0.2
_K = 3
_N = 39
_CIN = 2
_COUT = 64
_FC1 = 128
_FC2 = 256
_NCLS = 36
_PAD3 = 128


def _sgnet_kernel(T, E,
                  x_ref, srcr_ref, dstr_ref, dstc_ref,
                  wbd_ref, bconv_ref, w1_ref, b1_ref, w2_ref, b2_ref,
                  w3_ref, b3_ref, out_ref, conv_ref):
    f32 = jnp.float32
    NC = _N * _CIN

    src_row = srcr_ref[...]
    dst_row = dstr_ref[...]
    dst_col = dstc_ref[...]
    iota_ne = jax.lax.broadcasted_iota(jnp.int32, (_N, E), 0)
    s_ohT = (iota_ne == src_row).astype(f32)
    d_ohT = (iota_ne == dst_row).astype(f32)
    iota_en = jax.lax.broadcasted_iota(jnp.int32, (E, _N), 1)
    d_oh = (iota_en == dst_col).astype(f32)

    deg = jnp.sum(d_ohT, axis=1, keepdims=True)
    dinv = jnp.where(deg > 0, 1.0 / jnp.sqrt(deg), 0.0)
    dinv_src = jnp.sum(s_ohT * dinv, axis=0, keepdims=True)
    dinv_dst = jnp.sum(d_ohT * dinv, axis=0, keepdims=True)
    norm = dinv_src * dinv_dst

    arev = jnp.dot(s_ohT * norm, d_oh, preferred_element_type=f32)

    i0 = jax.lax.broadcasted_iota(jnp.int32, (NC, _N), 0)
    i1 = jax.lax.broadcasted_iota(jnp.int32, (NC, _N), 1)
    r_exp = ((i0 // 2) == i1).astype(f32)
    j0 = jax.lax.broadcasted_iota(jnp.int32, (_N, NC), 0)
    j1 = jax.lax.broadcasted_iota(jnp.int32, (_N, NC), 1)
    r_row = (j0 == (j1 // 2)).astype(f32)
    p0 = jax.lax.broadcasted_iota(jnp.int32, (NC, NC), 0)
    p1 = jax.lax.broadcasted_iota(jnp.int32, (NC, NC), 1)
    parity = ((p0 % 2) == (p1 % 2)).astype(f32)
    m_un = jnp.dot(jnp.dot(r_exp, arev, preferred_element_type=f32), r_row,
                   preferred_element_type=f32)
    m_prop = m_un * parity

    z0 = x_ref[...]
    z1 = jnp.dot(z0, m_prop, preferred_element_type=f32)
    z2 = jnp.dot(z1, m_prop, preferred_element_type=f32)
    z3 = jnp.dot(z2, m_prop, preferred_element_type=f32)
    xs = jnp.concatenate([z0, z1, z2, z3], axis=1)

    conv_ref[...] = (jnp.dot(xs, wbd_ref[...], preferred_element_type=f32)
                     + bconv_ref[...])

    w1 = w1_ref[...]
    b1 = b1_ref[...]
    w2 = w2_ref[...]
    b2 = b2_ref[...]
    w3 = w3_ref[...]
    b3 = b3_ref[...]

    def step(t, carry):
        (c1m, c1s, h1m, h1s, h2m, h2s, h3m, h3s, h3sum) = carry
        conv_t = conv_ref[pl.ds(t, 1), :]
        c1m = c1m * _DECAY * (1.0 - c1s) + conv_t
        c1s = (c1m > _THRESH).astype(f32)
        h1m = (h1m * _DECAY * (1.0 - h1s)
               + jnp.dot(c1s, w1, preferred_element_type=f32) + b1)
        h1s = (h1m > _THRESH).astype(f32)
        h2m = (h2m * _DECAY * (1.0 - h2s)
               + jnp.dot(h1s, w2, preferred_element_type=f32) + b2)
        h2s = (h2m > _THRESH).astype(f32)
        h3m = (h3m * _DECAY * (1.0 - h3s)
               + jnp.dot(h2s, w3, preferred_element_type=f32) + b3)
        h3s = (h3m > _THRESH).astype(f32)
        h3sum = h3sum + h3s
        return (c1m, c1s, h1m, h1s, h2m, h2s, h3m, h3s, h3sum)

    zf = functools.partial(jnp.zeros, dtype=f32)
    init = (zf((1, _N * _COUT)), zf((1, _N * _COUT)),
            zf((1, _FC1)), zf((1, _FC1)),
            zf((1, _FC2)), zf((1, _FC2)),
            zf((1, _PAD3)), zf((1, _PAD3)), zf((1, _PAD3)))
    final = jax.lax.fori_loop(0, T, step, init)
    out_ref[...] = final[8] * (1.0 / T)


def kernel(input, edge_index, W_conv, b_conv, W1, b1, W2, b2, W3, b3):
    T = input.shape[2]
    E = edge_index.shape[1]
    f32 = jnp.float32

    x_tm = input.transpose(2, 0, 1).reshape(T, _N * _CIN).astype(f32)

    ei = edge_index.astype(jnp.int32)
    src_row = ei[0].reshape(1, E)
    dst_row = ei[1].reshape(1, E)
    dst_col = ei[1].reshape(E, 1)

    wr = W_conv.reshape(_K + 1, _CIN, _COUT).astype(f32)
    eye_n = jnp.eye(_N, dtype=f32)
    wbd = (eye_n[None, :, None, :, None] * wr[:, None, :, None, :]).reshape(
        (_K + 1) * _N * _CIN, _N * _COUT)
    bconv_row = jnp.tile(b_conv.astype(f32), _N).reshape(1, _N * _COUT)

    w1t = W1.T.astype(f32)
    b1r = b1.reshape(1, _FC1).astype(f32)
    w2t = W2.T.astype(f32)
    b2r = b2.reshape(1, _FC2).astype(f32)
    w3p = jnp.zeros((_FC2, _PAD3), f32).at[:, :_NCLS].set(W3.T.astype(f32))
    b3p = jnp.zeros((1, _PAD3), f32).at[0, :_NCLS].set(b3.astype(f32))

    out = pl.pallas_call(
        functools.partial(_sgnet_kernel, T, E),
        out_shape=jax.ShapeDtypeStruct((1, _PAD3), f32),
        scratch_shapes=[pltpu.VMEM((T, _N * _COUT), f32)],
    )(x_tm, src_row, dst_row, dst_col,
      wbd, bconv_row, w1t, b1r, w2t, b2r, w3p, b3p)
    return out[0, :_NCLS]


# speedup vs baseline: 116.1216x; 1.1707x over previous
import functools

import jax
import jax.numpy as jnp
from jax.experimental import pallas as pl
from jax.experimental.pallas import tpu as pltpu

_THRESH = 0.5
_DECAY = 0.2
_K = 3
_N = 39
_CIN = 2
_COUT = 64
_FC1 = 128
_FC2 = 256
_NCLS = 36
_PAD3 = 128


def _sgnet_kernel(T, E,
                  x_ref, srcr_ref, dstr_ref, dstc_ref,
                  wbd_ref, bconv_ref, w1_ref, b1_ref, w2_ref, b2_ref,
                  w3_ref, b3_ref, out_ref, conv_ref):
    f32 = jnp.float32
    NC = _N * _CIN

    src_row = srcr_ref[...]
    dst_row = dstr_ref[...]
    dst_col = dstc_ref[...]
    iota_ne = jax.lax.broadcasted_iota(jnp.int32, (_N, E), 0)
    s_ohT = (iota_ne == src_row).astype(f32)
    d_ohT = (iota_ne == dst_row).astype(f32)
    iota_en = jax.lax.broadcasted_iota(jnp.int32, (E, _N), 1)
    d_oh = (iota_en == dst_col).astype(f32)

    deg = jnp.sum(d_ohT, axis=1, keepdims=True)
    dinv = jnp.where(deg > 0, 1.0 / jnp.sqrt(deg), 0.0)
    dinv_src = jnp.sum(s_ohT * dinv, axis=0, keepdims=True)
    dinv_dst = jnp.sum(d_ohT * dinv, axis=0, keepdims=True)
    norm = dinv_src * dinv_dst

    arev = jnp.dot(s_ohT * norm, d_oh, preferred_element_type=f32)

    i0 = jax.lax.broadcasted_iota(jnp.int32, (NC, _N), 0)
    i1 = jax.lax.broadcasted_iota(jnp.int32, (NC, _N), 1)
    r_exp = ((i0 // 2) == i1).astype(f32)
    j0 = jax.lax.broadcasted_iota(jnp.int32, (_N, NC), 0)
    j1 = jax.lax.broadcasted_iota(jnp.int32, (_N, NC), 1)
    r_row = (j0 == (j1 // 2)).astype(f32)
    p0 = jax.lax.broadcasted_iota(jnp.int32, (NC, NC), 0)
    p1 = jax.lax.broadcasted_iota(jnp.int32, (NC, NC), 1)
    parity = ((p0 % 2) == (p1 % 2)).astype(f32)
    m_un = jnp.dot(jnp.dot(r_exp, arev, preferred_element_type=f32), r_row,
                   preferred_element_type=f32)
    m_prop = m_un * parity

    z0 = x_ref[...]
    z1 = jnp.dot(z0, m_prop, preferred_element_type=f32)
    z2 = jnp.dot(z1, m_prop, preferred_element_type=f32)
    z3 = jnp.dot(z2, m_prop, preferred_element_type=f32)
    xs = jnp.concatenate([z0, z1, z2, z3], axis=1)

    conv_ref[...] = (jnp.dot(xs, wbd_ref[...], preferred_element_type=f32)
                     + bconv_ref[...])

    w1 = w1_ref[...]
    b1 = b1_ref[...]
    w2 = w2_ref[...]
    b2 = b2_ref[...]
    w3 = w3_ref[...]
    b3 = b3_ref[...]

    def step(t, carry):
        (c1m, c1s, h1m, h1s, h2m, h2s, h3m, h3s, h3sum) = carry
        conv_t = conv_ref[pl.ds(t, 1), :]
        c1m = c1m * _DECAY * (1.0 - c1s) + conv_t
        c1s = (c1m > _THRESH).astype(f32)
        h1m = (h1m * _DECAY * (1.0 - h1s)
               + jnp.dot(c1s, w1, preferred_element_type=f32) + b1)
        h1s = (h1m > _THRESH).astype(f32)
        h2m = (h2m * _DECAY * (1.0 - h2s)
               + jnp.dot(h1s, w2, preferred_element_type=f32) + b2)
        h2s = (h2m > _THRESH).astype(f32)
        h3m = (h3m * _DECAY * (1.0 - h3s)
               + jnp.dot(h2s, w3, preferred_element_type=f32) + b3)
        h3s = (h3m > _THRESH).astype(f32)
        h3sum = h3sum + h3s
        return (c1m, c1s, h1m, h1s, h2m, h2s, h3m, h3s, h3sum)

    def step4(i, carry):
        t = i * 4
        carry = step(t, carry)
        carry = step(t + 1, carry)
        carry = step(t + 2, carry)
        carry = step(t + 3, carry)
        return carry

    zf = functools.partial(jnp.zeros, dtype=f32)
    init = (zf((1, _N * _COUT)), zf((1, _N * _COUT)),
            zf((1, _FC1)), zf((1, _FC1)),
            zf((1, _FC2)), zf((1, _FC2)),
            zf((1, _PAD3)), zf((1, _PAD3)), zf((1, _PAD3)))
    if T % 4 == 0:
        final = jax.lax.fori_loop(0, T // 4, step4, init)
    else:
        final = jax.lax.fori_loop(0, T, step, init)
    out_ref[...] = final[8] * (1.0 / T)


def kernel(input, edge_index, W_conv, b_conv, W1, b1, W2, b2, W3, b3):
    T = input.shape[2]
    E = edge_index.shape[1]
    f32 = jnp.float32

    x_tm = input.transpose(2, 0, 1).reshape(T, _N * _CIN).astype(f32)

    ei = edge_index.astype(jnp.int32)
    src_row = ei[0].reshape(1, E)
    dst_row = ei[1].reshape(1, E)
    dst_col = ei[1].reshape(E, 1)

    wr = W_conv.reshape(_K + 1, _CIN, _COUT).astype(f32)
    eye_n = jnp.eye(_N, dtype=f32)
    wbd = (eye_n[None, :, None, :, None] * wr[:, None, :, None, :]).reshape(
        (_K + 1) * _N * _CIN, _N * _COUT)
    bconv_row = jnp.tile(b_conv.astype(f32), _N).reshape(1, _N * _COUT)

    w1t = W1.T.astype(f32)
    b1r = b1.reshape(1, _FC1).astype(f32)
    w2t = W2.T.astype(f32)
    b2r = b2.reshape(1, _FC2).astype(f32)
    w3p = jnp.zeros((_FC2, _PAD3), f32).at[:, :_NCLS].set(W3.T.astype(f32))
    b3p = jnp.zeros((1, _PAD3), f32).at[0, :_NCLS].set(b3.astype(f32))

    out = pl.pallas_call(
        functools.partial(_sgnet_kernel, T, E),
        out_shape=jax.ShapeDtypeStruct((1, _PAD3), f32),
        scratch_shapes=[pltpu.VMEM((T, _N * _COUT), f32)],
    )(x_tm, src_row, dst_row, dst_col,
      wbd, bconv_row, w1t, b1r, w2t, b2r, w3p, b3p)
    return out[0, :_NCLS]


# speedup vs baseline: 120.7764x; 1.0401x over previous
import functools

import jax
import jax.numpy as jnp
from jax.experimental import pallas as pl
from jax.experimental.pallas import tpu as pltpu

_THRESH = 0.5
_DECAY = 0.2
_K = 3
_N = 39
_CIN = 2
_COUT = 64
_FC1 = 128
_FC2 = 256
_NCLS = 36
_PAD3 = 128


def _sgnet_kernel(T, E,
                  x_ref, srcr_ref, dstr_ref, dstc_ref,
                  wbd_ref, bconv_ref, w1_ref, b1_ref, w2_ref, b2_ref,
                  w3_ref, b3_ref, out_ref, conv_ref):
    f32 = jnp.float32
    NC = _N * _CIN

    src_row = srcr_ref[...]
    dst_row = dstr_ref[...]
    dst_col = dstc_ref[...]
    iota_ne = jax.lax.broadcasted_iota(jnp.int32, (_N, E), 0)
    s_ohT = (iota_ne == src_row).astype(f32)
    d_ohT = (iota_ne == dst_row).astype(f32)
    iota_en = jax.lax.broadcasted_iota(jnp.int32, (E, _N), 1)
    d_oh = (iota_en == dst_col).astype(f32)

    deg = jnp.sum(d_ohT, axis=1, keepdims=True)
    dinv = jnp.where(deg > 0, 1.0 / jnp.sqrt(deg), 0.0)
    dinv_src = jnp.sum(s_ohT * dinv, axis=0, keepdims=True)
    dinv_dst = jnp.sum(d_ohT * dinv, axis=0, keepdims=True)
    norm = dinv_src * dinv_dst

    arev = jnp.dot(s_ohT * norm, d_oh, preferred_element_type=f32)

    i0 = jax.lax.broadcasted_iota(jnp.int32, (NC, _N), 0)
    i1 = jax.lax.broadcasted_iota(jnp.int32, (NC, _N), 1)
    r_exp = ((i0 // 2) == i1).astype(f32)
    j0 = jax.lax.broadcasted_iota(jnp.int32, (_N, NC), 0)
    j1 = jax.lax.broadcasted_iota(jnp.int32, (_N, NC), 1)
    r_row = (j0 == (j1 // 2)).astype(f32)
    p0 = jax.lax.broadcasted_iota(jnp.int32, (NC, NC), 0)
    p1 = jax.lax.broadcasted_iota(jnp.int32, (NC, NC), 1)
    parity = ((p0 % 2) == (p1 % 2)).astype(f32)
    m_un = jnp.dot(jnp.dot(r_exp, arev, preferred_element_type=f32), r_row,
                   preferred_element_type=f32)
    m_prop = m_un * parity

    z0 = x_ref[...]
    z1 = jnp.dot(z0, m_prop, preferred_element_type=f32)
    z2 = jnp.dot(z1, m_prop, preferred_element_type=f32)
    z3 = jnp.dot(z2, m_prop, preferred_element_type=f32)
    xs = jnp.concatenate([z0, z1, z2, z3], axis=1)

    conv_ref[...] = (jnp.dot(xs, wbd_ref[...], preferred_element_type=f32)
                     + bconv_ref[...])

    w1 = w1_ref[...]
    b1 = b1_ref[...]
    w2 = w2_ref[...]
    b2 = b2_ref[...]
    w3 = w3_ref[...]
    b3 = b3_ref[...]

    def step(t, carry):
        (c1m, c1s, h1m, h1s, h2m, h2s, h3m, h3s, h3sum) = carry
        conv_t = conv_ref[pl.ds(t, 1), :]
        c1m = c1m * _DECAY * (1.0 - c1s) + conv_t
        c1s = (c1m > _THRESH).astype(f32)
        h1m = (h1m * _DECAY * (1.0 - h1s)
               + jnp.dot(c1s, w1, preferred_element_type=f32) + b1)
        h1s = (h1m > _THRESH).astype(f32)
        h2m = (h2m * _DECAY * (1.0 - h2s)
               + jnp.dot(h1s, w2, preferred_element_type=f32) + b2)
        h2s = (h2m > _THRESH).astype(f32)
        h3m = (h3m * _DECAY * (1.0 - h3s)
               + jnp.dot(h2s, w3, preferred_element_type=f32) + b3)
        h3s = (h3m > _THRESH).astype(f32)
        h3sum = h3sum + h3s
        return (c1m, c1s, h1m, h1s, h2m, h2s, h3m, h3s, h3sum)

    _UNROLL = 10

    def step_u(i, carry):
        t = i * _UNROLL
        for u in range(_UNROLL):
            carry = step(t + u, carry)
        return carry

    zf = functools.partial(jnp.zeros, dtype=f32)
    init = (zf((1, _N * _COUT)), zf((1, _N * _COUT)),
            zf((1, _FC1)), zf((1, _FC1)),
            zf((1, _FC2)), zf((1, _FC2)),
            zf((1, _PAD3)), zf((1, _PAD3)), zf((1, _PAD3)))
    if T % _UNROLL == 0:
        final = jax.lax.fori_loop(0, T // _UNROLL, step_u, init)
    else:
        final = jax.lax.fori_loop(0, T, step, init)
    out_ref[...] = final[8] * (1.0 / T)


def kernel(input, edge_index, W_conv, b_conv, W1, b1, W2, b2, W3, b3):
    T = input.shape[2]
    E = edge_index.shape[1]
    f32 = jnp.float32

    x_tm = input.transpose(2, 0, 1).reshape(T, _N * _CIN).astype(f32)

    ei = edge_index.astype(jnp.int32)
    src_row = ei[0].reshape(1, E)
    dst_row = ei[1].reshape(1, E)
    dst_col = ei[1].reshape(E, 1)

    wr = W_conv.reshape(_K + 1, _CIN, _COUT).astype(f32)
    eye_n = jnp.eye(_N, dtype=f32)
    wbd = (eye_n[None, :, None, :, None] * wr[:, None, :, None, :]).reshape(
        (_K + 1) * _N * _CIN, _N * _COUT)
    bconv_row = jnp.tile(b_conv.astype(f32), _N).reshape(1, _N * _COUT)

    w1t = W1.T.astype(f32)
    b1r = b1.reshape(1, _FC1).astype(f32)
    w2t = W2.T.astype(f32)
    b2r = b2.reshape(1, _FC2).astype(f32)
    w3p = jnp.zeros((_FC2, _PAD3), f32).at[:, :_NCLS].set(W3.T.astype(f32))
    b3p = jnp.zeros((1, _PAD3), f32).at[0, :_NCLS].set(b3.astype(f32))

    out = pl.pallas_call(
        functools.partial(_sgnet_kernel, T, E),
        out_shape=jax.ShapeDtypeStruct((1, _PAD3), f32),
        scratch_shapes=[pltpu.VMEM((T, _N * _COUT), f32)],
    )(x_tm, src_row, dst_row, dst_col,
      wbd, bconv_row, w1t, b1r, w2t, b2r, w3p, b3p)
    return out[0, :_NCLS]


# speedup vs baseline: 254.1326x; 2.1042x over previous
import functools

import jax
import jax.numpy as jnp
from jax.experimental import pallas as pl
from jax.experimental.pallas import tpu as pltpu

_THRESH = 0.5
_DECAY = 0.2
_K = 3
_N = 39
_CIN = 2
_COUT = 64
_FC1 = 128
_FC2 = 256
_NCLS = 36
_PAD3 = 128


def _sgnet_kernel(T, E,
                  x_ref, srcr_ref, dstr_ref, dstc_ref,
                  wbd_ref, bconv_ref, w1_ref, b1_ref, w2_ref, b2_ref,
                  w3_ref, b3_ref, out_ref,
                  conv_ref, c1s_ref, h1in_ref, h1s_ref, h2in_ref, h2s_ref,
                  h3in_ref):
    f32 = jnp.float32
    NC = _N * _CIN

    src_row = srcr_ref[...]
    dst_row = dstr_ref[...]
    dst_col = dstc_ref[...]
    iota_ne = jax.lax.broadcasted_iota(jnp.int32, (_N, E), 0)
    s_ohT = (iota_ne == src_row).astype(f32)
    d_ohT = (iota_ne == dst_row).astype(f32)
    iota_en = jax.lax.broadcasted_iota(jnp.int32, (E, _N), 1)
    d_oh = (iota_en == dst_col).astype(f32)

    deg = jnp.sum(d_ohT, axis=1, keepdims=True)
    dinv = jnp.where(deg > 0, 1.0 / jnp.sqrt(deg), 0.0)
    dinv_src = jnp.sum(s_ohT * dinv, axis=0, keepdims=True)
    dinv_dst = jnp.sum(d_ohT * dinv, axis=0, keepdims=True)
    norm = dinv_src * dinv_dst

    arev = jnp.dot(s_ohT * norm, d_oh, preferred_element_type=f32)

    i0 = jax.lax.broadcasted_iota(jnp.int32, (NC, _N), 0)
    i1 = jax.lax.broadcasted_iota(jnp.int32, (NC, _N), 1)
    r_exp = ((i0 // 2) == i1).astype(f32)
    j0 = jax.lax.broadcasted_iota(jnp.int32, (_N, NC), 0)
    j1 = jax.lax.broadcasted_iota(jnp.int32, (_N, NC), 1)
    r_row = (j0 == (j1 // 2)).astype(f32)
    p0 = jax.lax.broadcasted_iota(jnp.int32, (NC, NC), 0)
    p1 = jax.lax.broadcasted_iota(jnp.int32, (NC, NC), 1)
    parity = ((p0 % 2) == (p1 % 2)).astype(f32)
    m_un = jnp.dot(jnp.dot(r_exp, arev, preferred_element_type=f32), r_row,
                   preferred_element_type=f32)
    m_prop = m_un * parity

    z0 = x_ref[...]
    z1 = jnp.dot(z0, m_prop, preferred_element_type=f32)
    z2 = jnp.dot(z1, m_prop, preferred_element_type=f32)
    z3 = jnp.dot(z2, m_prop, preferred_element_type=f32)
    xs = jnp.concatenate([z0, z1, z2, z3], axis=1)

    conv_ref[...] = (jnp.dot(xs, wbd_ref[...], preferred_element_type=f32)
                     + bconv_ref[...])

    def scan_c1(t, carry):
        m, s = carry
        m = m * _DECAY * (1.0 - s) + conv_ref[pl.ds(t, 1), :]
        s = (m > _THRESH).astype(f32)
        c1s_ref[pl.ds(t, 1), :] = s
        return (m, s)

    zf = functools.partial(jnp.zeros, dtype=f32)
    jax.lax.fori_loop(0, T, scan_c1,
                      (zf((1, _N * _COUT)), zf((1, _N * _COUT))))

    h1in_ref[...] = (jnp.dot(c1s_ref[...], w1_ref[...],
                             preferred_element_type=f32) + b1_ref[...])

    def scan_h1(t, carry):
        m, s = carry
        m = m * _DECAY * (1.0 - s) + h1in_ref[pl.ds(t, 1), :]
        s = (m > _THRESH).astype(f32)
        h1s_ref[pl.ds(t, 1), :] = s
        return (m, s)

    jax.lax.fori_loop(0, T, scan_h1, (zf((1, _FC1)), zf((1, _FC1))))

    h2in_ref[...] = (jnp.dot(h1s_ref[...], w2_ref[...],
                             preferred_element_type=f32) + b2_ref[...])

    def scan_h2(t, carry):
        m, s = carry
        m = m * _DECAY * (1.0 - s) + h2in_ref[pl.ds(t, 1), :]
        s = (m > _THRESH).astype(f32)
        h2s_ref[pl.ds(t, 1), :] = s
        return (m, s)

    jax.lax.fori_loop(0, T, scan_h2, (zf((1, _FC2)), zf((1, _FC2))))

    h3in_ref[...] = (jnp.dot(h2s_ref[...], w3_ref[...],
                             preferred_element_type=f32) + b3_ref[...])

    def scan_h3(t, carry):
        m, s, acc = carry
        m = m * _DECAY * (1.0 - s) + h3in_ref[pl.ds(t, 1), :]
        s = (m > _THRESH).astype(f32)
        return (m, s, acc + s)

    _, _, h3sum = jax.lax.fori_loop(
        0, T, scan_h3, (zf((1, _PAD3)), zf((1, _PAD3)), zf((1, _PAD3))))
    out_ref[...] = h3sum * (1.0 / T)


def kernel(input, edge_index, W_conv, b_conv, W1, b1, W2, b2, W3, b3):
    T = input.shape[2]
    E = edge_index.shape[1]
    f32 = jnp.float32

    x_tm = input.transpose(2, 0, 1).reshape(T, _N * _CIN).astype(f32)

    ei = edge_index.astype(jnp.int32)
    src_row = ei[0].reshape(1, E)
    dst_row = ei[1].reshape(1, E)
    dst_col = ei[1].reshape(E, 1)

    wr = W_conv.reshape(_K + 1, _CIN, _COUT).astype(f32)
    eye_n = jnp.eye(_N, dtype=f32)
    wbd = (eye_n[None, :, None, :, None] * wr[:, None, :, None, :]).reshape(
        (_K + 1) * _N * _CIN, _N * _COUT)
    bconv_row = jnp.tile(b_conv.astype(f32), _N).reshape(1, _N * _COUT)

    w1t = W1.T.astype(f32)
    b1r = b1.reshape(1, _FC1).astype(f32)
    w2t = W2.T.astype(f32)
    b2r = b2.reshape(1, _FC2).astype(f32)
    w3p = jnp.zeros((_FC2, _PAD3), f32).at[:, :_NCLS].set(W3.T.astype(f32))
    b3p = jnp.zeros((1, _PAD3), f32).at[0, :_NCLS].set(b3.astype(f32))

    out = pl.pallas_call(
        functools.partial(_sgnet_kernel, T, E),
        out_shape=jax.ShapeDtypeStruct((1, _PAD3), f32),
        scratch_shapes=[
            pltpu.VMEM((T, _N * _COUT), f32),
            pltpu.VMEM((T, _N * _COUT), f32),
            pltpu.VMEM((T, _FC1), f32),
            pltpu.VMEM((T, _FC1), f32),
            pltpu.VMEM((T, _FC2), f32),
            pltpu.VMEM((T, _FC2), f32),
            pltpu.VMEM((T, _PAD3), f32),
        ],
    )(x_tm, src_row, dst_row, dst_col,
      wbd, bconv_row, w1t, b1r, w2t, b2r, w3p, b3p)
    return out[0, :_NCLS]


# speedup vs baseline: 263.9829x; 1.0388x over previous
import functools

import jax
import jax.numpy as jnp
from jax.experimental import pallas as pl
from jax.experimental.pallas import tpu as pltpu

_THRESH = 0.5
_DECAY = 0.2
_K = 3
_N = 39
_CIN = 2
_COUT = 64
_FC1 = 128
_FC2 = 256
_NCLS = 36
_PAD3 = 128


def _sgnet_kernel(T, E,
                  x_ref, srcr_ref, dstr_ref, dstc_ref,
                  wbd_ref, bconv_ref, w1_ref, b1_ref, w2_ref, b2_ref,
                  w3_ref, b3_ref, out_ref,
                  conv_ref, c1s_ref, h1in_ref, h1s_ref, h2in_ref, h2s_ref,
                  h3in_ref):
    f32 = jnp.float32
    NC = _N * _CIN

    src_row = srcr_ref[...]
    dst_row = dstr_ref[...]
    dst_col = dstc_ref[...]
    iota_ne = jax.lax.broadcasted_iota(jnp.int32, (_N, E), 0)
    s_ohT = (iota_ne == src_row).astype(f32)
    d_ohT = (iota_ne == dst_row).astype(f32)
    iota_en = jax.lax.broadcasted_iota(jnp.int32, (E, _N), 1)
    d_oh = (iota_en == dst_col).astype(f32)

    deg = jnp.sum(d_ohT, axis=1, keepdims=True)
    dinv = jnp.where(deg > 0, 1.0 / jnp.sqrt(deg), 0.0)
    dinv_src = jnp.sum(s_ohT * dinv, axis=0, keepdims=True)
    dinv_dst = jnp.sum(d_ohT * dinv, axis=0, keepdims=True)
    norm = dinv_src * dinv_dst

    arev = jnp.dot(s_ohT * norm, d_oh, preferred_element_type=f32)

    i0 = jax.lax.broadcasted_iota(jnp.int32, (NC, _N), 0)
    i1 = jax.lax.broadcasted_iota(jnp.int32, (NC, _N), 1)
    r_exp = ((i0 // 2) == i1).astype(f32)
    j0 = jax.lax.broadcasted_iota(jnp.int32, (_N, NC), 0)
    j1 = jax.lax.broadcasted_iota(jnp.int32, (_N, NC), 1)
    r_row = (j0 == (j1 // 2)).astype(f32)
    p0 = jax.lax.broadcasted_iota(jnp.int32, (NC, NC), 0)
    p1 = jax.lax.broadcasted_iota(jnp.int32, (NC, NC), 1)
    parity = ((p0 % 2) == (p1 % 2)).astype(f32)
    m_un = jnp.dot(jnp.dot(r_exp, arev, preferred_element_type=f32), r_row,
                   preferred_element_type=f32)
    m_prop = m_un * parity

    z0 = x_ref[...]
    z1 = jnp.dot(z0, m_prop, preferred_element_type=f32)
    z2 = jnp.dot(z1, m_prop, preferred_element_type=f32)
    z3 = jnp.dot(z2, m_prop, preferred_element_type=f32)
    xs = jnp.concatenate([z0, z1, z2, z3], axis=1)

    conv_ref[...] = (jnp.dot(xs, wbd_ref[...], preferred_element_type=f32)
                     + bconv_ref[...])

    UNROLL = 10 if T % 10 == 0 else 1

    def run_scan(body, init):
        if UNROLL == 1:
            return jax.lax.fori_loop(0, T, body, init)

        def unrolled(i, carry):
            t = i * UNROLL
            for u in range(UNROLL):
                carry = body(t + u, carry)
            return carry

        return jax.lax.fori_loop(0, T // UNROLL, unrolled, init)

    def scan_c1(t, carry):
        m, s = carry
        m = m * _DECAY * (1.0 - s) + conv_ref[pl.ds(t, 1), :]
        s = (m > _THRESH).astype(f32)
        c1s_ref[pl.ds(t, 1), :] = s
        return (m, s)

    zf = functools.partial(jnp.zeros, dtype=f32)
    run_scan(scan_c1, (zf((1, _N * _COUT)), zf((1, _N * _COUT))))

    h1in_ref[...] = (jnp.dot(c1s_ref[...], w1_ref[...],
                             preferred_element_type=f32) + b1_ref[...])

    def scan_h1(t, carry):
        m, s = carry
        m = m * _DECAY * (1.0 - s) + h1in_ref[pl.ds(t, 1), :]
        s = (m > _THRESH).astype(f32)
        h1s_ref[pl.ds(t, 1), :] = s
        return (m, s)

    run_scan(scan_h1, (zf((1, _FC1)), zf((1, _FC1))))

    h2in_ref[...] = (jnp.dot(h1s_ref[...], w2_ref[...],
                             preferred_element_type=f32) + b2_ref[...])

    def scan_h2(t, carry):
        m, s = carry
        m = m * _DECAY * (1.0 - s) + h2in_ref[pl.ds(t, 1), :]
        s = (m > _THRESH).astype(f32)
        h2s_ref[pl.ds(t, 1), :] = s
        return (m, s)

    run_scan(scan_h2, (zf((1, _FC2)), zf((1, _FC2))))

    h3in_ref[...] = (jnp.dot(h2s_ref[...], w3_ref[...],
                             preferred_element_type=f32) + b3_ref[...])

    def scan_h3(t, carry):
        m, s, acc = carry
        m = m * _DECAY * (1.0 - s) + h3in_ref[pl.ds(t, 1), :]
        s = (m > _THRESH).astype(f32)
        return (m, s, acc + s)

    _, _, h3sum = run_scan(
        scan_h3, (zf((1, _PAD3)), zf((1, _PAD3)), zf((1, _PAD3))))
    out_ref[...] = h3sum * (1.0 / T)


def kernel(input, edge_index, W_conv, b_conv, W1, b1, W2, b2, W3, b3):
    T = input.shape[2]
    E = edge_index.shape[1]
    f32 = jnp.float32

    x_tm = input.transpose(2, 0, 1).reshape(T, _N * _CIN).astype(f32)

    ei = edge_index.astype(jnp.int32)
    src_row = ei[0].reshape(1, E)
    dst_row = ei[1].reshape(1, E)
    dst_col = ei[1].reshape(E, 1)

    wr = W_conv.reshape(_K + 1, _CIN, _COUT).astype(f32)
    eye_n = jnp.eye(_N, dtype=f32)
    wbd = (eye_n[None, :, None, :, None] * wr[:, None, :, None, :]).reshape(
        (_K + 1) * _N * _CIN, _N * _COUT)
    bconv_row = jnp.tile(b_conv.astype(f32), _N).reshape(1, _N * _COUT)

    w1t = W1.T.astype(f32)
    b1r = b1.reshape(1, _FC1).astype(f32)
    w2t = W2.T.astype(f32)
    b2r = b2.reshape(1, _FC2).astype(f32)
    w3p = jnp.zeros((_FC2, _PAD3), f32).at[:, :_NCLS].set(W3.T.astype(f32))
    b3p = jnp.zeros((1, _PAD3), f32).at[0, :_NCLS].set(b3.astype(f32))

    out = pl.pallas_call(
        functools.partial(_sgnet_kernel, T, E),
        out_shape=jax.ShapeDtypeStruct((1, _PAD3), f32),
        scratch_shapes=[
            pltpu.VMEM((T, _N * _COUT), f32),
            pltpu.VMEM((T, _N * _COUT), f32),
            pltpu.VMEM((T, _FC1), f32),
            pltpu.VMEM((T, _FC1), f32),
            pltpu.VMEM((T, _FC2), f32),
            pltpu.VMEM((T, _FC2), f32),
            pltpu.VMEM((T, _PAD3), f32),
        ],
    )(x_tm, src_row, dst_row, dst_col,
      wbd, bconv_row, w1t, b1r, w2t, b2r, w3p, b3p)
    return out[0, :_NCLS]


# speedup vs baseline: 390.5820x; 1.4796x over previous
import functools

import jax
import jax.numpy as jnp
from jax.experimental import pallas as pl
from jax.experimental.pallas import tpu as pltpu

_THRESH = 0.5
_DECAY = 0.2
_K = 3
_N = 39
_CIN = 2
_COUT = 64
_FC1 = 128
_FC2 = 256
_NCLS = 36
_PAD3 = 128


def _nt_dot(a, b):
    return jax.lax.dot_general(a, b, (((1,), (1,)), ((), ())),
                               preferred_element_type=jnp.float32)


def _sgnet_kernel(T, E,
                  x_ref, srcr_ref, dstr_ref, dstc_ref,
                  wconv_ref, bconv_ref, w1_ref, b1_ref, w2_ref, b2_ref,
                  w3_ref, b3_ref, out_ref,
                  conv_ref, c1s_ref, h1in_ref, h1s_ref, h2in_ref, h2s_ref,
                  h3in_ref):
    f32 = jnp.float32
    NC = _N * _CIN

    src_row = srcr_ref[...]
    dst_row = dstr_ref[...]
    dst_col = dstc_ref[...]
    iota_ne = jax.lax.broadcasted_iota(jnp.int32, (_N, E), 0)
    s_ohT = (iota_ne == src_row).astype(f32)
    d_ohT = (iota_ne == dst_row).astype(f32)
    iota_en = jax.lax.broadcasted_iota(jnp.int32, (E, _N), 1)
    d_oh = (iota_en == dst_col).astype(f32)

    deg = jnp.sum(d_ohT, axis=1, keepdims=True)
    dinv = jnp.where(deg > 0, 1.0 / jnp.sqrt(deg), 0.0)
    dinv_src = jnp.sum(s_ohT * dinv, axis=0, keepdims=True)
    dinv_dst = jnp.sum(d_ohT * dinv, axis=0, keepdims=True)
    norm = dinv_src * dinv_dst

    arev = jnp.dot(s_ohT * norm, d_oh, preferred_element_type=f32)

    i0 = jax.lax.broadcasted_iota(jnp.int32, (NC, _N), 0)
    i1 = jax.lax.broadcasted_iota(jnp.int32, (NC, _N), 1)
    r_exp = ((i0 // 2) == i1).astype(f32)
    j0 = jax.lax.broadcasted_iota(jnp.int32, (_N, NC), 0)
    j1 = jax.lax.broadcasted_iota(jnp.int32, (_N, NC), 1)
    r_row = (j0 == (j1 // 2)).astype(f32)
    p0 = jax.lax.broadcasted_iota(jnp.int32, (NC, NC), 0)
    p1 = jax.lax.broadcasted_iota(jnp.int32, (NC, NC), 1)
    parity = ((p0 % 2) == (p1 % 2)).astype(f32)
    m_un = jnp.dot(jnp.dot(r_exp, arev, preferred_element_type=f32), r_row,
                   preferred_element_type=f32)
    m_prop = m_un * parity

    z0 = x_ref[...]
    z1 = jnp.dot(z0, m_prop, preferred_element_type=f32)
    z2 = jnp.dot(z1, m_prop, preferred_element_type=f32)
    z3 = jnp.dot(z2, m_prop, preferred_element_type=f32)
    xs = jnp.concatenate([z0, z1, z2, z3], axis=1)

    P = (_K + 1) * _CIN
    NP = (_K + 1) * NC
    e0 = jax.lax.broadcasted_iota(jnp.int32, (NP, P), 0)
    e1 = jax.lax.broadcasted_iota(jnp.int32, (NP, P), 1)
    esel = (e1 == ((e0 // NC) * _CIN + e0 % _CIN)).astype(f32)
    wsel = jnp.dot(esel, wconv_ref[...], preferred_element_type=f32)
    wtile = pltpu.repeat(wsel, _N, axis=1)
    m0 = jax.lax.broadcasted_iota(jnp.int32, (NP, _N * _COUT), 0)
    m1 = jax.lax.broadcasted_iota(jnp.int32, (NP, _N * _COUT), 1)
    nmask = (((m0 // _CIN) % _N) == (m1 // _COUT)).astype(f32)
    wbd = wtile * nmask
    bconv_row = pltpu.repeat(bconv_ref[...], _N, axis=1)

    conv_ref[...] = jnp.dot(xs, wbd, preferred_element_type=f32) + bconv_row

    UNROLL = 10 if T % 10 == 0 else 1

    def run_scan(body, init):
        if UNROLL == 1:
            return jax.lax.fori_loop(0, T, body, init)

        def unrolled(i, carry):
            t = i * UNROLL
            for u in range(UNROLL):
                carry = body(t + u, carry)
            return carry

        return jax.lax.fori_loop(0, T // UNROLL, unrolled, init)

    def scan_c1(t, carry):
        m, s = carry
        m = m * _DECAY * (1.0 - s) + conv_ref[pl.ds(t, 1), :]
        s = (m > _THRESH).astype(f32)
        c1s_ref[pl.ds(t, 1), :] = s
        return (m, s)

    zf = functools.partial(jnp.zeros, dtype=f32)
    run_scan(scan_c1, (zf((1, _N * _COUT)), zf((1, _N * _COUT))))

    h1in_ref[...] = _nt_dot(c1s_ref[...], w1_ref[...]) + b1_ref[...]

    def scan_h1(t, carry):
        m, s = carry
        m = m * _DECAY * (1.0 - s) + h1in_ref[pl.ds(t, 1), :]
        s = (m > _THRESH).astype(f32)
        h1s_ref[pl.ds(t, 1), :] = s
        return (m, s)

    run_scan(scan_h1, (zf((1, _FC1)), zf((1, _FC1))))

    h2in_ref[...] = _nt_dot(h1s_ref[...], w2_ref[...]) + b2_ref[...]

    def scan_h2(t, carry):
        m, s = carry
        m = m * _DECAY * (1.0 - s) + h2in_ref[pl.ds(t, 1), :]
        s = (m > _THRESH).astype(f32)
        h2s_ref[pl.ds(t, 1), :] = s
        return (m, s)

    run_scan(scan_h2, (zf((1, _FC2)), zf((1, _FC2))))

    h3in_ref[...] = _nt_dot(h2s_ref[...], w3_ref[...]) + b3_ref[...]

    def scan_h3(t, carry):
        m, s, acc = carry
        m = m * _DECAY * (1.0 - s) + h3in_ref[pl.ds(t, 1), :]
        s = (m > _THRESH).astype(f32)
        return (m, s, acc + s)

    _, _, h3sum = run_scan(
        scan_h3, (zf((1, _PAD3)), zf((1, _PAD3)), zf((1, _PAD3))))
    out_ref[...] = h3sum * (1.0 / T)


def kernel(input, edge_index, W_conv, b_conv, W1, b1, W2, b2, W3, b3):
    T = input.shape[2]
    E = edge_index.shape[1]
    f32 = jnp.float32

    x_tm = input.transpose(2, 0, 1).reshape(T, _N * _CIN).astype(f32)

    ei = edge_index.astype(jnp.int32)
    src_row = ei[0].reshape(1, E)
    dst_row = ei[1].reshape(1, E)
    dst_col = ei[1].reshape(E, 1)

    wconv = W_conv.astype(f32)
    bconv = b_conv.reshape(1, _COUT).astype(f32)
    b1r = b1.reshape(1, _FC1).astype(f32)
    b2r = b2.reshape(1, _FC2).astype(f32)
    w3p = jnp.zeros((_PAD3, _FC2), f32).at[:_NCLS, :].set(W3.astype(f32))
    b3p = jnp.zeros((1, _PAD3), f32).at[0, :_NCLS].set(b3.astype(f32))

    out = pl.pallas_call(
        functools.partial(_sgnet_kernel, T, E),
        out_shape=jax.ShapeDtypeStruct((1, _PAD3), f32),
        scratch_shapes=[
            pltpu.VMEM((T, _N * _COUT), f32),
            pltpu.VMEM((T, _N * _COUT), f32),
            pltpu.VMEM((T, _FC1), f32),
            pltpu.VMEM((T, _FC1), f32),
            pltpu.VMEM((T, _FC2), f32),
            pltpu.VMEM((T, _FC2), f32),
            pltpu.VMEM((T, _PAD3), f32),
        ],
    )(x_tm, src_row, dst_row, dst_col,
      wconv, bconv, W1.astype(f32), b1r, W2.astype(f32), b2r, w3p, b3p)
    return out[0, :_NCLS]
